# Initial kernel scaffold; baseline (speedup 1.0000x reference)
import jax
import jax.numpy as jnp
from jax.experimental import pallas as pl


def kernel(indices, weight):
    raise NotImplementedError("write your pallas kernel here")


# speedup vs baseline: 1.0948x; 1.0948x over previous
import functools

import jax
import jax.numpy as jnp
from jax import lax
from jax.experimental import pallas as pl
from jax.experimental.pallas import tpu as pltpu
from jax.experimental.pallas import tpu_sc as plsc

_CHUNK = 1024


def _make_gather(B, V, D):
    info = plsc.get_sparse_core_info()
    NW = info.num_cores * info.num_subcores
    b_per_w = B // NW
    n_chunks = b_per_w // _CHUNK
    assert b_per_w % _CHUNK == 0

    mesh = plsc.VectorSubcoreMesh(core_axis_name="c", subcore_axis_name="s")

    @functools.partial(
        pl.kernel,
        mesh=mesh,
        out_type=jax.ShapeDtypeStruct((B, D), jnp.float32),
        scratch_types=[
            pltpu.VMEM((_CHUNK,), jnp.int32),
            pltpu.VMEM((_CHUNK, D), jnp.float32),
            pltpu.SemaphoreType.DMA,
        ],
        compiler_params=pltpu.CompilerParams(use_tc_tiling_on_sc=False),
    )
    def gather_kernel(idx_hbm, table_hbm, out_hbm, idx_v, rows_v, sem):
        wid = lax.axis_index("s") * info.num_cores + lax.axis_index("c")
        base = wid * b_per_w

        def step(c, carry):
            off = base + c * _CHUNK
            pltpu.sync_copy(idx_hbm.at[pl.ds(off, _CHUNK)], idx_v)
            pltpu.async_copy(table_hbm.at[idx_v], rows_v, sem).wait()
            pltpu.sync_copy(rows_v, out_hbm.at[pl.ds(off, _CHUNK)])
            return carry

        lax.fori_loop(0, n_chunks, step, 0)

    return gather_kernel


def kernel(indices, weight):
    B0, B1 = indices.shape
    V, D = weight.shape
    B = B0 * B1
    flat_idx = indices.reshape(B).astype(jnp.int32)
    out = _make_gather(B, V, D)(flat_idx, weight)
    return out.reshape(B0, B1, D)


# speedup vs baseline: 1.1121x; 1.0157x over previous
import functools

import jax
import jax.numpy as jnp
from jax import lax
from jax.experimental import pallas as pl
from jax.experimental.pallas import tpu as pltpu
from jax.experimental.pallas import tpu_sc as plsc

_CHUNK = 800
_K = 4


def _make_gather(B, V, D):
    info = plsc.get_sparse_core_info()
    NW = info.num_cores * info.num_subcores
    b_per_w = B // NW
    n_groups = b_per_w // (_K * _CHUNK)
    assert b_per_w % (_K * _CHUNK) == 0

    mesh = plsc.VectorSubcoreMesh(core_axis_name="c", subcore_axis_name="s")

    @functools.partial(
        pl.kernel,
        mesh=mesh,
        out_type=jax.ShapeDtypeStruct((B, D), jnp.float32),
        scratch_types=[
            pltpu.VMEM((b_per_w,), jnp.int32),
            pltpu.VMEM((_K, _CHUNK, D), jnp.float32),
            pltpu.SemaphoreType.DMA,
            pltpu.SemaphoreType.DMA,
        ],
        compiler_params=pltpu.CompilerParams(use_tc_tiling_on_sc=False),
    )
    def gather_kernel(idx_hbm, table_hbm, out_hbm, idx_v, rows_v, gsem, osem):
        wid = lax.axis_index("s") * info.num_cores + lax.axis_index("c")
        base = wid * b_per_w
        pltpu.sync_copy(idx_hbm.at[pl.ds(base, b_per_w)], idx_v)

        def group(g, carry):
            goff = pl.multiple_of(g * (_K * _CHUNK), _K * _CHUNK)
            gdescs = []
            for b in range(_K):
                idx_sl = idx_v.at[pl.ds(goff + b * _CHUNK, _CHUNK)]
                gdescs.append(
                    pltpu.async_copy(table_hbm.at[idx_sl], rows_v.at[b], gsem))
            sdescs = []
            for b in range(_K):
                gdescs[b].wait()
                out_sl = out_hbm.at[pl.ds(base + goff + b * _CHUNK, _CHUNK)]
                sdescs.append(pltpu.async_copy(rows_v.at[b], out_sl, osem))
            for b in range(_K):
                sdescs[b].wait()
            return carry

        lax.fori_loop(0, n_groups, group, 0)

    return gather_kernel


def kernel(indices, weight):
    B0, B1 = indices.shape
    V, D = weight.shape
    B = B0 * B1
    flat_idx = indices.reshape(B).astype(jnp.int32)
    out = _make_gather(B, V, D)(flat_idx, weight)
    return out.reshape(B0, B1, D)


# speedup vs baseline: 1.6150x; 1.4522x over previous
import functools

import jax
import jax.numpy as jnp
from jax import lax
from jax.experimental import pallas as pl
from jax.experimental.pallas import tpu as pltpu
from jax.experimental.pallas import tpu_sc as plsc

_B0, _B1, _D = 16384, 50, 32
_JPW = 4
_GB1 = 10
_NG = _JPW * (_B1 // _GB1)
_GROWS = _GB1 * 128


def _make_gather(B, V):
    info = plsc.get_sparse_core_info()
    NC = info.num_cores
    NW = NC * info.num_subcores
    b_per_w = B // NW

    mesh = plsc.VectorSubcoreMesh(core_axis_name="c", subcore_axis_name="s")

    @functools.partial(
        pl.kernel,
        mesh=mesh,
        out_type=jax.ShapeDtypeStruct((_B1, _D // 8, _B0 // 128, 8, 128),
                                      jnp.float32),
        scratch_types=[
            pltpu.VMEM((b_per_w,), jnp.int32),
            pltpu.VMEM((2, _GROWS), jnp.int32),
            pltpu.VMEM((2, _GROWS, _D), jnp.float32),
            pltpu.VMEM((2, _D // 8, 8, 128), jnp.float32),
            pltpu.SemaphoreType.DMA,
            pltpu.SemaphoreType.DMA,
            pltpu.SemaphoreType.DMA,
            pltpu.SemaphoreType.DMA,
        ],
        compiler_params=pltpu.CompilerParams(use_tc_tiling_on_sc=False, needs_layout_passes=False),
    )
    def gather_kernel(idx_hbm, table_hbm, out_hbm, idx_all, gi, rows, ob,
                      gsem0, gsem1, osem0, osem1):
        wid = lax.axis_index("s") * NC + lax.axis_index("c")
        base = wid * b_per_w
        pltpu.sync_copy(idx_hbm.at[pl.ds(base, b_per_w)], idx_all)

        iota = lax.iota(jnp.int32, 16)
        gsems = (gsem0, gsem1)
        osems = (osem0, osem1)

        def extract(slot, g):
            j_local = g // (_B1 // _GB1)
            b1_0 = (g % (_B1 // _GB1)) * _GB1
            for t in range(_GB1):
                for lc in range(8):
                    src = (j_local * (128 * _B1) + b1_0 + t
                           + (lc * 16) * _B1) + iota * _B1
                    v = plsc.load_gather(idx_all, [src])
                    dst = jnp.full((16,), t * 128 + lc * 16, jnp.int32) + iota
                    plsc.store_scatter(
                        gi, [jnp.full((16,), slot, jnp.int32), dst], v)

        def issue(slot):
            return pltpu.async_copy(table_hbm.at[gi.at[slot]], rows.at[slot],
                                    gsems[slot])

        def drain_gather(slot):
            pltpu.make_async_copy(table_hbm.at[pl.ds(0, _GROWS)],
                                  rows.at[slot], gsems[slot]).wait()

        def drain_store(slot):
            pltpu.make_async_copy(
                table_hbm.at[pl.ds(0, (_D // 8) * 8 * 128 // _D)],
                ob.at[slot], osems[slot]).wait()

        extract(0, 0)
        issue(0)

        def outer(p, carry):
            for q in (0, 1):
                g = p * 2 + q
                j_local = g // (_B1 // _GB1)
                b1_0 = (g % (_B1 // _GB1)) * _GB1
                jglob = wid * _JPW + j_local

                @pl.when(g < _NG - 1)
                def _():
                    extract(1 - q, g + 1)
                    issue(1 - q)

                drain_gather(q)

                def tloop(t5, tc):
                    for sub in (0, 1):
                        t = t5 * 2 + sub
                        b1 = b1_0 + t

                        @pl.when(g * _GB1 + t >= 2)
                        def _():
                            drain_store(sub)

                        qv = jnp.full((16,), q, jnp.int32)
                        for lc in range(8):
                            rowv = (jnp.full((16,), lc * 16, jnp.int32)
                                    + t * 128 + iota)
                            for c in range(_D):
                                v = plsc.load_gather(
                                    rows, [qv, rowv,
                                           jnp.full((16,), c, jnp.int32)])
                                plsc.store_scatter(
                                    ob,
                                    [jnp.full((16,), sub, jnp.int32),
                                     jnp.full((16,), c // 8, jnp.int32),
                                     jnp.full((16,), c % 8, jnp.int32),
                                     jnp.full((16,), lc * 16, jnp.int32) + iota],
                                    v)
                        pltpu.async_copy(ob.at[sub],
                                         out_hbm.at[b1, :, jglob],
                                         osems[sub])
                    return tc

                lax.fori_loop(0, _GB1 // 2, tloop, 0)
            return carry

        lax.fori_loop(0, _NG // 2, outer, 0)
        drain_store(0)
        drain_store(1)

    return gather_kernel


def kernel(indices, weight):
    B0, B1 = indices.shape
    V, D = weight.shape
    B = B0 * B1
    flat_idx = indices.reshape(B).astype(jnp.int32)
    out5 = _make_gather(B, V)(flat_idx, weight)
    return out5.transpose(2, 4, 0, 1, 3).reshape(B0, B1, D)


# speedup vs baseline: 1.6185x; 1.0022x over previous
import functools

import jax
import jax.numpy as jnp
from jax import lax
from jax.experimental import pallas as pl
from jax.experimental.pallas import tpu as pltpu
from jax.experimental.pallas import tpu_sc as plsc

_B0, _B1, _D = 16384, 50, 32
_JPW = 4
_GB1 = 10
_NG = _JPW * (_B1 // _GB1)
_GROWS = _GB1 * 128


def _make_gather(B, V):
    info = plsc.get_sparse_core_info()
    NC = info.num_cores
    NW = NC * info.num_subcores
    b_per_w = B // NW

    mesh = plsc.VectorSubcoreMesh(core_axis_name="c", subcore_axis_name="s")

    @functools.partial(
        pl.kernel,
        mesh=mesh,
        out_type=jax.ShapeDtypeStruct((_B1, _D // 8, _B0 // 128, 8, 128),
                                      jnp.float32),
        scratch_types=[
            pltpu.VMEM((b_per_w,), jnp.int32),
            pltpu.VMEM((2, _GROWS), jnp.int32),
            pltpu.VMEM((2 * _GROWS, _D), jnp.float32),
            pltpu.VMEM((2, _D // 8, 8, 128), jnp.float32),
            pltpu.SemaphoreType.DMA,
            pltpu.SemaphoreType.DMA,
            pltpu.SemaphoreType.DMA,
            pltpu.SemaphoreType.DMA,
        ],
        compiler_params=pltpu.CompilerParams(use_tc_tiling_on_sc=False,
                                             needs_layout_passes=False),
    )
    def gather_kernel(idx_hbm, table_hbm, out_hbm, idx_all, gi, rows, ob,
                      gsem0, gsem1, osem0, osem1):
        wid = lax.axis_index("s") * NC + lax.axis_index("c")
        base = wid * b_per_w
        pltpu.sync_copy(idx_hbm.at[pl.ds(base, b_per_w)], idx_all)

        iota = lax.iota(jnp.int32, 16)
        gsems = (gsem0, gsem1)
        osems = (osem0, osem1)

        def extract(slot, g):
            j_local = g // (_B1 // _GB1)
            b1_0 = (g % (_B1 // _GB1)) * _GB1
            sbase = j_local * (128 * _B1) + b1_0
            for t in range(_GB1):
                for lc in range(8):
                    src = (sbase + (t + lc * 16 * _B1)) + iota * _B1
                    v = plsc.load_gather(idx_all, [src])
                    gi[slot, pl.ds(t * 128 + lc * 16, 16)] = v

        def issue(slot):
            pltpu.async_copy(table_hbm.at[gi.at[slot]],
                             rows.at[pl.ds(slot * _GROWS, _GROWS)],
                             gsems[slot])

        def drain_gather(slot):
            pltpu.make_async_copy(table_hbm.at[pl.ds(0, _GROWS)],
                                  rows.at[pl.ds(0, _GROWS)],
                                  gsems[slot]).wait()

        def drain_store(slot):
            pltpu.make_async_copy(
                table_hbm.at[pl.ds(0, (_D // 8) * 8 * 128 // _D)],
                ob.at[slot], osems[slot]).wait()

        extract(0, 0)
        issue(0)

        def outer(p, carry):
            for q in (0, 1):
                g = p * 2 + q
                j_local = g // (_B1 // _GB1)
                b1_0 = (g % (_B1 // _GB1)) * _GB1
                jglob = wid * _JPW + j_local

                @pl.when(g < _NG - 1)
                def _():
                    extract(1 - q, g + 1)
                    issue(1 - q)

                drain_gather(q)
                rq = q * _GROWS

                def tloop(t5, tc):
                    for sub in (0, 1):
                        t = t5 * 2 + sub
                        b1 = b1_0 + t

                        @pl.when(g * _GB1 + t >= 2)
                        def _():
                            drain_store(sub)

                        rbase0 = rq + t * 128
                        for lc in range(8):
                            rowv = (rbase0 + lc * 16) + iota
                            for c in range(_D):
                                v = plsc.load_gather(
                                    rows, [rowv, jnp.full((16,), c, jnp.int32)])
                                ob[sub, c // 8, c % 8, pl.ds(lc * 16, 16)] = v
                        pltpu.async_copy(ob.at[sub],
                                         out_hbm.at[b1, :, jglob],
                                         osems[sub])
                    return tc

                lax.fori_loop(0, _GB1 // 2, tloop, 0)
            return carry

        lax.fori_loop(0, _NG // 2, outer, 0)
        drain_store(0)
        drain_store(1)

    return gather_kernel


def kernel(indices, weight):
    B0, B1 = indices.shape
    V, D = weight.shape
    B = B0 * B1
    flat_idx = indices.reshape(B).astype(jnp.int32)
    out5 = _make_gather(B, V)(flat_idx, weight)
    return out5.transpose(2, 4, 0, 1, 3).reshape(B0, B1, D)
